# baseline (speedup 1.0000x reference)
import functools

import jax
import jax.numpy as jnp
from jax.experimental import pallas as pl
from jax.experimental.pallas import tpu as pltpu

K_NN = 32
_BIG = float(2 ** 30)


def _knn_kernel(q_ref, k_ref, v_ref, o_ref, topd_ref, topv_ref,
                lm_ref, li_ref, done_ref, *, n_keys, chunk, n_chunks, nwin):
    c = pl.program_id(0)
    wlen = chunk // nwin

    @pl.when(c == 0)
    def _init():
        topd_ref[...] = jnp.full_like(topd_ref, jnp.inf)
        topv_ref[...] = jnp.zeros_like(topv_ref)

    q = q_ref[...]
    k = k_ref[...]
    vb = v_ref[0]

    qsq = jnp.sum(q * q, axis=1, keepdims=True)
    ksq = jnp.sum(k * k, axis=1)[None, :]
    qk = jax.lax.dot_general(q, k, (((1,), (1,)), ((), ())),
                             preferred_element_type=jnp.float32)
    dist = qsq - 2.0 * qk + ksq

    lane = jax.lax.broadcasted_iota(jnp.int32, (1, chunk), 1).astype(jnp.float32)
    valid = (jnp.float32(c * chunk) + lane) < n_keys
    dist = jnp.where(valid, dist, jnp.inf)

    slot = jax.lax.broadcasted_iota(jnp.int32, topd_ref.shape, 1).astype(jnp.float32)

    dws = [dist[:, w * wlen:(w + 1) * wlen] for w in range(nwin)]
    vbs = [vb[:, w * wlen:(w + 1) * wlen] for w in range(nwin)]
    lanew = lane[:, :wlen]

    lm_ref[...] = jnp.full_like(lm_ref, -jnp.inf)
    li_ref[...] = jnp.full_like(li_ref, -1.0)
    done_ref[0] = 0

    def body(j, carry):
        @pl.when(done_ref[0] == 0)
        def _pass():
            ms, idxs, vals = [], [], []
            for w in range(nwin):
                lmw = lm_ref[:, w:w + 1]
                liw = li_ref[:, w:w + 1]
                dw = dws[w]
                lexw = (dw > lmw) | ((dw == lmw) & (lanew > liw))
                mw = jnp.min(jnp.where(lexw, dw, jnp.inf), axis=1, keepdims=True)
                idxw = jnp.min(jnp.where((dw == mw) & lexw, lanew, _BIG),
                               axis=1, keepdims=True)
                valw = jnp.sum(jnp.where(lanew == idxw, vbs[w], 0.0),
                               axis=1, keepdims=True)
                ms.append(mw)
                idxs.append(idxw)
                vals.append(valw)
            lm_ref[...] = jnp.concatenate(ms, axis=1)
            li_ref[...] = jnp.concatenate(idxs, axis=1)

            topd = topd_ref[...]
            topv = topv_ref[...]
            nav = jnp.zeros((topd.shape[0], 1), jnp.float32)
            for w in range(nwin):
                t = jnp.max(topd, axis=1, keepdims=True)
                acc = ms[w] < t
                pos = jnp.min(jnp.where(topd == t, slot, jnp.float32(64.0)),
                              axis=1, keepdims=True)
                repl = (slot == pos) & acc
                topd = jnp.where(repl, ms[w], topd)
                topv = jnp.where(repl, vals[w], topv)
                nav = nav + acc.astype(jnp.float32)
            topd_ref[...] = topd
            topv_ref[...] = topv
            na = jnp.sum(nav)

            @pl.when(na == 0.0)
            def _done():
                done_ref[0] = 1

        return carry

    jax.lax.fori_loop(0, K_NN + 1, body, 0)

    @pl.when(c == n_chunks - 1)
    def _emit():
        o_ref[...] = jnp.sum(topv_ref[...], axis=1, keepdims=True) / jnp.float32(K_NN)


def kernel(queries, keys, values):
    n_queries, dim = queries.shape
    n_keys = keys.shape[0]
    chunk = 2048
    nwin = 8
    n_chunks = pl.cdiv(n_keys, chunk)
    n_pad = n_chunks * chunk

    keys_p = jnp.pad(keys, ((0, n_pad - n_keys), (0, 0)))
    values_p = jnp.pad(values, (0, n_pad - n_keys)).reshape(n_chunks, 1, chunk)

    out = pl.pallas_call(
        functools.partial(_knn_kernel, n_keys=n_keys, chunk=chunk,
                          n_chunks=n_chunks, nwin=nwin),
        grid=(n_chunks,),
        in_specs=[
            pl.BlockSpec((n_queries, dim), lambda c: (0, 0)),
            pl.BlockSpec((chunk, dim), lambda c: (c, 0)),
            pl.BlockSpec((1, 1, chunk), lambda c: (c, 0, 0)),
        ],
        out_specs=pl.BlockSpec((n_queries, 1), lambda c: (0, 0)),
        out_shape=jax.ShapeDtypeStruct((n_queries, 1), jnp.float32),
        scratch_shapes=[
            pltpu.VMEM((n_queries, K_NN), jnp.float32),
            pltpu.VMEM((n_queries, K_NN), jnp.float32),
            pltpu.VMEM((n_queries, nwin), jnp.float32),
            pltpu.VMEM((n_queries, nwin), jnp.float32),
            pltpu.SMEM((1,), jnp.int32),
        ],
    )(queries, keys_p, values_p)
    return out[:, 0]


# speedup vs baseline: 1.7825x; 1.7825x over previous
import functools

import jax
import jax.numpy as jnp
from jax.experimental import pallas as pl
from jax.experimental.pallas import tpu as pltpu

K_NN = 32
_LW = 128
_BIG = float(2 ** 30)
_IBASE = float(2 ** 22)


def _knn_kernel(q_ref, k_ref, v_ref, o_ref, dist_ref, b_ref, g_ref, vv_ref,
                topd_ref, topi_ref, topv_ref, flags_ref,
                *, n_keys, chunk, n_chunks):
    c = pl.program_id(0)
    ncol = chunk // _LW

    @pl.when(c == 0)
    def _init():
        topd_ref[...] = jnp.full_like(topd_ref, jnp.inf)
        topi_ref[...] = (jax.lax.broadcasted_iota(jnp.int32, topi_ref.shape, 1)
                         .astype(jnp.float32) + _IBASE)
        topv_ref[...] = jnp.zeros_like(topv_ref)

    q = q_ref[...]
    k = k_ref[...]
    vb = v_ref[0]

    qsq = jnp.sum(q * q, axis=1, keepdims=True)
    ksq = jnp.sum(k * k, axis=1)[None, :]
    qk = jax.lax.dot_general(q, k, (((1,), (1,)), ((), ())),
                             preferred_element_type=jnp.float32)
    dist = qsq - 2.0 * qk + ksq

    lane_c = jax.lax.broadcasted_iota(jnp.int32, (1, chunk), 1).astype(jnp.float32)
    valid = (jnp.float32(c * chunk) + lane_c) < n_keys
    dist_ref[...] = jnp.where(valid, dist, jnp.inf)

    lane = jax.lax.broadcasted_iota(jnp.int32, (1, _LW), 1).astype(jnp.float32)
    flags_ref[0] = 0

    def round_body(r, carry_r):
        @pl.when(flags_ref[0] == 0)
        def _round():
            cols = [dist_ref[:, j * _LW:(j + 1) * _LW] for j in range(ncol)]
            mins = cols
            while len(mins) > 1:
                nxt = [jnp.minimum(mins[2 * i], mins[2 * i + 1])
                       for i in range(len(mins) // 2)]
                if len(mins) % 2:
                    nxt.append(mins[-1])
                mins = nxt
            bb = mins[0]
            ml = jnp.full(bb.shape, float(ncol), jnp.float32)
            for j in reversed(range(ncol)):
                ml = jnp.where(cols[j] == bb, jnp.float32(j), ml)
            gg = ml * jnp.float32(_LW) + lane + jnp.float32(c * chunk)

            topd = topd_ref[...]
            topi = topi_ref[...]
            t = jnp.max(topd, axis=1, keepdims=True)
            ti = jnp.max(jnp.where(topd == t, topi, -1.0), axis=1, keepdims=True)
            cand = (bb < t) | ((bb == t) & (gg < ti))
            cnt = jnp.sum(cand.astype(jnp.float32))

            @pl.when(cnt == 0.0)
            def _done():
                flags_ref[0] = 1

            @pl.when(cnt > 0.0)
            def _merge():
                vvv = jnp.zeros(bb.shape, jnp.float32)
                for j in range(ncol):
                    vcol = vb[:, j * _LW:(j + 1) * _LW]
                    isj = ml == jnp.float32(j)
                    vvv = jnp.where(isj, vcol, vvv)
                    dist_ref[:, j * _LW:(j + 1) * _LW] = jnp.where(
                        isj & (cols[j] == bb), jnp.inf, cols[j])
                b_ref[...] = bb
                g_ref[...] = gg
                vv_ref[...] = vvv
                flags_ref[1] = 0

                def ins_body(i, carry_i):
                    @pl.when(flags_ref[1] == 0)
                    def _ins():
                        bcur = b_ref[...]
                        gcur = g_ref[...]
                        bm = jnp.min(bcur, axis=1, keepdims=True)
                        gm = jnp.min(jnp.where(bcur == bm, gcur, _BIG),
                                     axis=1, keepdims=True)
                        sel = (bcur == bm) & (gcur == gm)
                        val = jnp.sum(jnp.where(sel, vv_ref[...], 0.0),
                                      axis=1, keepdims=True)
                        td = topd_ref[...]
                        tix = topi_ref[...]
                        tt = jnp.max(td, axis=1, keepdims=True)
                        tti = jnp.max(jnp.where(td == tt, tix, -1.0),
                                      axis=1, keepdims=True)
                        acc = (bm < tt) | ((bm == tt) & (gm < tti))
                        na = jnp.sum(acc.astype(jnp.float32))

                        @pl.when(na == 0.0)
                        def _idone():
                            flags_ref[1] = 1

                        @pl.when(na > 0.0)
                        def _do():
                            repl = (td == tt) & (tix == tti) & acc
                            topd_ref[...] = jnp.where(repl, bm, td)
                            topi_ref[...] = jnp.where(repl, gm, tix)
                            topv_ref[...] = jnp.where(repl, val, topv_ref[...])
                            b_ref[...] = jnp.where(sel, jnp.inf, bcur)

                    return carry_i

                jax.lax.fori_loop(0, K_NN, ins_body, 0)

        return carry_r

    jax.lax.fori_loop(0, K_NN, round_body, 0)

    @pl.when(c == n_chunks - 1)
    def _emit():
        o_ref[...] = jnp.sum(topv_ref[...], axis=1, keepdims=True) / jnp.float32(K_NN)


def kernel(queries, keys, values):
    n_queries, dim = queries.shape
    n_keys = keys.shape[0]
    chunk = 2048
    n_chunks = pl.cdiv(n_keys, chunk)
    n_pad = n_chunks * chunk

    keys_p = jnp.pad(keys, ((0, n_pad - n_keys), (0, 0)))
    values_p = jnp.pad(values, (0, n_pad - n_keys)).reshape(n_chunks, 1, chunk)

    out = pl.pallas_call(
        functools.partial(_knn_kernel, n_keys=n_keys, chunk=chunk,
                          n_chunks=n_chunks),
        grid=(n_chunks,),
        in_specs=[
            pl.BlockSpec((n_queries, dim), lambda c: (0, 0)),
            pl.BlockSpec((chunk, dim), lambda c: (c, 0)),
            pl.BlockSpec((1, 1, chunk), lambda c: (c, 0, 0)),
        ],
        out_specs=pl.BlockSpec((n_queries, 1), lambda c: (0, 0)),
        out_shape=jax.ShapeDtypeStruct((n_queries, 1), jnp.float32),
        scratch_shapes=[
            pltpu.VMEM((n_queries, chunk), jnp.float32),
            pltpu.VMEM((n_queries, _LW), jnp.float32),
            pltpu.VMEM((n_queries, _LW), jnp.float32),
            pltpu.VMEM((n_queries, _LW), jnp.float32),
            pltpu.VMEM((n_queries, K_NN), jnp.float32),
            pltpu.VMEM((n_queries, K_NN), jnp.float32),
            pltpu.VMEM((n_queries, K_NN), jnp.float32),
            pltpu.SMEM((2,), jnp.int32),
        ],
    )(queries, keys_p, values_p)
    return out[:, 0]


# speedup vs baseline: 2.2794x; 1.2788x over previous
import functools

import jax
import jax.numpy as jnp
from jax.experimental import pallas as pl
from jax.experimental.pallas import tpu as pltpu

K_NN = 32
_LW = 128
_BIG = float(2 ** 30)
_IBASE = float(2 ** 22)


def _knn_kernel(q_ref, k_ref, v_ref, o_ref, dist_ref, b_ref, g_ref, vv_ref,
                topd_ref, topi_ref, topv_ref, flags_ref,
                *, n_keys, chunk, n_chunks):
    c = pl.program_id(0)
    ncol = chunk // _LW

    @pl.when(c == 0)
    def _init():
        topd_ref[...] = jnp.full_like(topd_ref, jnp.inf)
        topi_ref[...] = (jax.lax.broadcasted_iota(jnp.int32, topi_ref.shape, 1)
                         .astype(jnp.float32) + _IBASE)
        topv_ref[...] = jnp.zeros_like(topv_ref)

    q = q_ref[...]
    k = k_ref[...]
    vb = v_ref[0]

    qsq = jnp.sum(q * q, axis=1, keepdims=True)
    ones = jnp.ones((1, k.shape[1]), jnp.float32)
    ksq = jax.lax.dot_general(ones, k * k, (((1,), (1,)), ((), ())),
                              preferred_element_type=jnp.float32)
    qk = jax.lax.dot_general(q, k, (((1,), (1,)), ((), ())),
                             preferred_element_type=jnp.float32)
    dist_ref[...] = qsq - 2.0 * qk + ksq

    n_tail = n_keys - (n_chunks - 1) * chunk

    @pl.when(c == n_chunks - 1)
    def _mask_tail():
        dist_ref[:, n_tail:] = jnp.full(
            (dist_ref.shape[0], chunk - n_tail), jnp.inf, jnp.float32)

    lane = jax.lax.broadcasted_iota(jnp.int32, (1, _LW), 1).astype(jnp.float32)
    flags_ref[0] = 0

    def round_body(r, carry_r):
        @pl.when(flags_ref[0] == 0)
        def _round():
            cols = [dist_ref[:, j * _LW:(j + 1) * _LW] for j in range(ncol)]
            mins = cols
            while len(mins) > 1:
                nxt = [jnp.minimum(mins[2 * i], mins[2 * i + 1])
                       for i in range(len(mins) // 2)]
                if len(mins) % 2:
                    nxt.append(mins[-1])
                mins = nxt
            bb = mins[0]
            ml = jnp.full(bb.shape, float(ncol), jnp.float32)
            for j in reversed(range(ncol)):
                ml = jnp.where(cols[j] == bb, jnp.float32(j), ml)
            gg = ml * jnp.float32(_LW) + lane + jnp.float32(c * chunk)

            topd = topd_ref[...]
            topi = topi_ref[...]
            t = jnp.max(topd, axis=1, keepdims=True)
            ti = jnp.max(jnp.where(topd == t, topi, -1.0), axis=1, keepdims=True)
            cand = (bb < t) | ((bb == t) & (gg < ti))
            cnt = jnp.sum(cand.astype(jnp.float32))

            @pl.when(cnt == 0.0)
            def _done():
                flags_ref[0] = 1

            @pl.when(cnt > 0.0)
            def _merge():
                vvv = jnp.zeros(bb.shape, jnp.float32)
                for j in range(ncol):
                    vcol = vb[:, j * _LW:(j + 1) * _LW]
                    isj = ml == jnp.float32(j)
                    vvv = jnp.where(isj, vcol, vvv)
                    dist_ref[:, j * _LW:(j + 1) * _LW] = jnp.where(
                        isj & (cols[j] == bb), jnp.inf, cols[j])
                b_ref[...] = bb
                g_ref[...] = gg
                vv_ref[...] = vvv
                flags_ref[1] = 0

                def ins_body(i, carry_i):
                    @pl.when(flags_ref[1] == 0)
                    def _ins():
                        bcur = b_ref[...]
                        gcur = g_ref[...]
                        bm = jnp.min(bcur, axis=1, keepdims=True)
                        gm = jnp.min(jnp.where(bcur == bm, gcur, _BIG),
                                     axis=1, keepdims=True)
                        sel = (bcur == bm) & (gcur == gm)
                        val = jnp.sum(jnp.where(sel, vv_ref[...], 0.0),
                                      axis=1, keepdims=True)
                        td = topd_ref[...]
                        tix = topi_ref[...]
                        tt = jnp.max(td, axis=1, keepdims=True)
                        tti = jnp.max(jnp.where(td == tt, tix, -1.0),
                                      axis=1, keepdims=True)
                        acc = (bm < tt) | ((bm == tt) & (gm < tti))
                        na = jnp.sum(acc.astype(jnp.float32))

                        @pl.when(na == 0.0)
                        def _idone():
                            flags_ref[1] = 1

                        @pl.when(na > 0.0)
                        def _do():
                            repl = (td == tt) & (tix == tti) & acc
                            topd_ref[...] = jnp.where(repl, bm, td)
                            topi_ref[...] = jnp.where(repl, gm, tix)
                            topv_ref[...] = jnp.where(repl, val, topv_ref[...])
                            b_ref[...] = jnp.where(sel, jnp.inf, bcur)

                    return carry_i

                jax.lax.fori_loop(0, K_NN, ins_body, 0)

        return carry_r

    jax.lax.fori_loop(0, K_NN, round_body, 0)

    @pl.when(c == n_chunks - 1)
    def _emit():
        o_ref[...] = jnp.sum(topv_ref[...], axis=1, keepdims=True) / jnp.float32(K_NN)


def kernel(queries, keys, values):
    n_queries, dim = queries.shape
    n_keys = keys.shape[0]
    chunk = 4096
    n_chunks = pl.cdiv(n_keys, chunk)
    n_pad = n_chunks * chunk

    keys_p = jnp.pad(keys, ((0, n_pad - n_keys), (0, 0)))
    values_p = jnp.pad(values, (0, n_pad - n_keys)).reshape(n_chunks, 1, chunk)

    out = pl.pallas_call(
        functools.partial(_knn_kernel, n_keys=n_keys, chunk=chunk,
                          n_chunks=n_chunks),
        grid=(n_chunks,),
        in_specs=[
            pl.BlockSpec((n_queries, dim), lambda c: (0, 0)),
            pl.BlockSpec((chunk, dim), lambda c: (c, 0)),
            pl.BlockSpec((1, 1, chunk), lambda c: (c, 0, 0)),
        ],
        out_specs=pl.BlockSpec((n_queries, 1), lambda c: (0, 0)),
        out_shape=jax.ShapeDtypeStruct((n_queries, 1), jnp.float32),
        scratch_shapes=[
            pltpu.VMEM((n_queries, chunk), jnp.float32),
            pltpu.VMEM((n_queries, _LW), jnp.float32),
            pltpu.VMEM((n_queries, _LW), jnp.float32),
            pltpu.VMEM((n_queries, _LW), jnp.float32),
            pltpu.VMEM((n_queries, K_NN), jnp.float32),
            pltpu.VMEM((n_queries, K_NN), jnp.float32),
            pltpu.VMEM((n_queries, K_NN), jnp.float32),
            pltpu.SMEM((2,), jnp.int32),
        ],
    )(queries, keys_p, values_p)
    return out[:, 0]


# speedup vs baseline: 2.2977x; 1.0080x over previous
import functools

import jax
import jax.numpy as jnp
from jax.experimental import pallas as pl
from jax.experimental.pallas import tpu as pltpu

K_NN = 32
_LW = 128
_BIG = float(2 ** 30)
_IBASE = float(2 ** 22)


def _knn_kernel(q_ref, k_ref, v_ref, o_ref, dist_ref, b_ref, g_ref, vv_ref,
                topd_ref, topi_ref, topv_ref, flags_ref,
                *, n_keys, chunk, n_chunks):
    c = pl.program_id(0)
    ncol = chunk // _LW

    @pl.when(c == 0)
    def _init():
        topd_ref[...] = jnp.full_like(topd_ref, jnp.inf)
        topi_ref[...] = (jax.lax.broadcasted_iota(jnp.int32, topi_ref.shape, 1)
                         .astype(jnp.float32) + _IBASE)
        topv_ref[...] = jnp.zeros_like(topv_ref)

    q = q_ref[...]
    k = k_ref[...]
    vb = v_ref[0]

    qsq = jnp.sum(q * q, axis=1, keepdims=True)
    ksq = jnp.sum(k * k, axis=1)[None, :]
    qk = jax.lax.dot_general(q, k, (((1,), (1,)), ((), ())),
                             preferred_element_type=jnp.float32)
    dist_ref[...] = qsq - 2.0 * qk + ksq

    n_tail = n_keys - (n_chunks - 1) * chunk

    @pl.when(c == n_chunks - 1)
    def _mask_tail():
        dist_ref[:, n_tail:] = jnp.full(
            (dist_ref.shape[0], chunk - n_tail), jnp.inf, jnp.float32)

    lane = jax.lax.broadcasted_iota(jnp.int32, (1, _LW), 1).astype(jnp.float32)
    flags_ref[0] = 0

    def round_body(r, carry_r):
        @pl.when(flags_ref[0] == 0)
        def _round():
            cols = [dist_ref[:, j * _LW:(j + 1) * _LW] for j in range(ncol)]
            mins = cols
            while len(mins) > 1:
                nxt = [jnp.minimum(mins[2 * i], mins[2 * i + 1])
                       for i in range(len(mins) // 2)]
                if len(mins) % 2:
                    nxt.append(mins[-1])
                mins = nxt
            bb = mins[0]
            ml = jnp.full(bb.shape, float(ncol), jnp.float32)
            for j in reversed(range(ncol)):
                ml = jnp.where(cols[j] == bb, jnp.float32(j), ml)
            gg = ml * jnp.float32(_LW) + lane + jnp.float32(c * chunk)

            topd = topd_ref[...]
            topi = topi_ref[...]
            t = jnp.max(topd, axis=1, keepdims=True)
            ti = jnp.max(jnp.where(topd == t, topi, -1.0), axis=1, keepdims=True)
            cand = (bb < t) | ((bb == t) & (gg < ti))
            cnt = jnp.sum(cand.astype(jnp.float32))

            @pl.when(cnt == 0.0)
            def _done():
                flags_ref[0] = 1

            @pl.when(cnt > 0.0)
            def _merge():
                vvv = jnp.zeros(bb.shape, jnp.float32)
                for j in range(ncol):
                    vcol = vb[:, j * _LW:(j + 1) * _LW]
                    isj = ml == jnp.float32(j)
                    vvv = jnp.where(isj, vcol, vvv)
                    dist_ref[:, j * _LW:(j + 1) * _LW] = jnp.where(
                        isj & (cols[j] == bb), jnp.inf, cols[j])
                b_ref[...] = bb
                g_ref[...] = gg
                vv_ref[...] = vvv
                flags_ref[1] = 0

                def ins_body(i, carry_i):
                    @pl.when(flags_ref[1] == 0)
                    def _ins():
                        bcur = b_ref[...]
                        gcur = g_ref[...]
                        bm = jnp.min(bcur, axis=1, keepdims=True)
                        gm = jnp.min(jnp.where(bcur == bm, gcur, _BIG),
                                     axis=1, keepdims=True)
                        sel = (bcur == bm) & (gcur == gm)
                        val = jnp.sum(jnp.where(sel, vv_ref[...], 0.0),
                                      axis=1, keepdims=True)
                        td = topd_ref[...]
                        tix = topi_ref[...]
                        tt = jnp.max(td, axis=1, keepdims=True)
                        tti = jnp.max(jnp.where(td == tt, tix, -1.0),
                                      axis=1, keepdims=True)
                        acc = (bm < tt) | ((bm == tt) & (gm < tti))
                        na = jnp.sum(acc.astype(jnp.float32))

                        @pl.when(na == 0.0)
                        def _idone():
                            flags_ref[1] = 1

                        @pl.when(na > 0.0)
                        def _do():
                            repl = (td == tt) & (tix == tti) & acc
                            topd_ref[...] = jnp.where(repl, bm, td)
                            topi_ref[...] = jnp.where(repl, gm, tix)
                            topv_ref[...] = jnp.where(repl, val, topv_ref[...])
                            b_ref[...] = jnp.where(sel, jnp.inf, bcur)

                    return carry_i

                jax.lax.fori_loop(0, K_NN, ins_body, 0)

        return carry_r

    jax.lax.fori_loop(0, K_NN, round_body, 0)

    @pl.when(c == n_chunks - 1)
    def _emit():
        o_ref[...] = jnp.sum(topv_ref[...], axis=1, keepdims=True) / jnp.float32(K_NN)


def kernel(queries, keys, values):
    n_queries, dim = queries.shape
    n_keys = keys.shape[0]
    chunk = 4096
    n_chunks = pl.cdiv(n_keys, chunk)
    n_pad = n_chunks * chunk

    keys_p = jnp.pad(keys, ((0, n_pad - n_keys), (0, 0)))
    values_p = jnp.pad(values, (0, n_pad - n_keys)).reshape(n_chunks, 1, chunk)

    out = pl.pallas_call(
        functools.partial(_knn_kernel, n_keys=n_keys, chunk=chunk,
                          n_chunks=n_chunks),
        grid=(n_chunks,),
        in_specs=[
            pl.BlockSpec((n_queries, dim), lambda c: (0, 0)),
            pl.BlockSpec((chunk, dim), lambda c: (c, 0)),
            pl.BlockSpec((1, 1, chunk), lambda c: (c, 0, 0)),
        ],
        out_specs=pl.BlockSpec((n_queries, 1), lambda c: (0, 0)),
        out_shape=jax.ShapeDtypeStruct((n_queries, 1), jnp.float32),
        scratch_shapes=[
            pltpu.VMEM((n_queries, chunk), jnp.float32),
            pltpu.VMEM((n_queries, _LW), jnp.float32),
            pltpu.VMEM((n_queries, _LW), jnp.float32),
            pltpu.VMEM((n_queries, _LW), jnp.float32),
            pltpu.VMEM((n_queries, K_NN), jnp.float32),
            pltpu.VMEM((n_queries, K_NN), jnp.float32),
            pltpu.VMEM((n_queries, K_NN), jnp.float32),
            pltpu.SMEM((2,), jnp.int32),
        ],
    )(queries, keys_p, values_p)
    return out[:, 0]


# speedup vs baseline: 2.5290x; 1.1007x over previous
import functools

import jax
import jax.numpy as jnp
from jax.experimental import pallas as pl
from jax.experimental.pallas import tpu as pltpu

K_NN = 32
_LW = 128
_BIG = float(2 ** 30)
_IBASE = float(2 ** 22)
_PAD = 1.0e15


def _batch_reduce(cols, lane, gbase):
    mins = cols
    while len(mins) > 1:
        nxt = [jnp.minimum(mins[2 * i], mins[2 * i + 1])
               for i in range(len(mins) // 2)]
        if len(mins) % 2:
            nxt.append(mins[-1])
        mins = nxt
    bb = mins[0]
    ml = jnp.full(bb.shape, float(len(cols)), jnp.float32)
    for j in reversed(range(len(cols))):
        ml = jnp.where(cols[j] == bb, jnp.float32(j), ml)
    gg = ml * jnp.float32(_LW) + lane + gbase
    return bb, ml, gg


def _top_state(topd_ref, topi_ref):
    topd = topd_ref[...]
    topi = topi_ref[...]
    t = jnp.max(topd, axis=1, keepdims=True)
    ti = jnp.max(jnp.where(topd == t, topi, -1.0), axis=1, keepdims=True)
    return t, ti


def _knn_kernel(q_ref, k_ref, v_ref, o_ref, dist_ref, b_ref, g_ref, vv_ref,
                topd_ref, topi_ref, topv_ref, flags_ref,
                *, n_keys, chunk, n_chunks):
    c = pl.program_id(0)
    ncol = chunk // _LW

    @pl.when(c == 0)
    def _init():
        topd_ref[...] = jnp.full_like(topd_ref, jnp.inf)
        topi_ref[...] = (jax.lax.broadcasted_iota(jnp.int32, topi_ref.shape, 1)
                         .astype(jnp.float32) + _IBASE)
        topv_ref[...] = jnp.zeros_like(topv_ref)

    q = q_ref[...]
    k = k_ref[...]
    vb = v_ref[0]

    qsq = jnp.sum(q * q, axis=1, keepdims=True)
    ksq = jnp.sum(k * k, axis=1)[None, :]
    qk = jax.lax.dot_general(q, k, (((1,), (1,)), ((), ())),
                             preferred_element_type=jnp.float32)
    dist = qsq - 2.0 * qk + ksq
    dist_ref[...] = dist

    lane = jax.lax.broadcasted_iota(jnp.int32, (1, _LW), 1).astype(jnp.float32)
    gbase = jnp.float32(c * chunk)

    def run_insertions():
        ub = flags_ref[1]

        def ins_body(i, carry_i):
            bcur = b_ref[...]
            gcur = g_ref[...]
            bm = jnp.min(bcur, axis=1, keepdims=True)
            gm = jnp.min(jnp.where(bcur == bm, gcur, _BIG),
                         axis=1, keepdims=True)
            sel = (bcur == bm) & (gcur == gm)
            val = jnp.sum(jnp.where(sel, vv_ref[...], 0.0),
                          axis=1, keepdims=True)
            td = topd_ref[...]
            tix = topi_ref[...]
            tt = jnp.max(td, axis=1, keepdims=True)
            tti = jnp.max(jnp.where(td == tt, tix, -1.0),
                          axis=1, keepdims=True)
            acc = (bm < tt) | ((bm == tt) & (gm < tti))
            repl = (td == tt) & (tix == tti) & acc
            topd_ref[...] = jnp.where(repl, bm, td)
            topi_ref[...] = jnp.where(repl, gm, tix)
            topv_ref[...] = jnp.where(repl, val, topv_ref[...])
            b_ref[...] = jnp.where(sel, jnp.inf, bcur)
            return carry_i

        jax.lax.fori_loop(0, ub, ins_body, 0)

    def extract_values(ml):
        vvv = jnp.zeros((q.shape[0], _LW), jnp.float32)
        for j in range(ncol):
            vvv = jnp.where(ml == jnp.float32(j),
                            vb[:, j * _LW:(j + 1) * _LW], vvv)
        return vvv

    cols = [dist[:, j * _LW:(j + 1) * _LW] for j in range(ncol)]
    bb, ml, gg = _batch_reduce(cols, lane, gbase)
    t, ti = _top_state(topd_ref, topi_ref)
    cand = (bb < t) | ((bb == t) & (gg < ti))
    cnq = jnp.sum(cand.astype(jnp.float32), axis=1, keepdims=True)
    flags_ref[1] = jnp.minimum(jnp.max(cnq), float(K_NN + 1)).astype(jnp.int32)
    cnt2 = jnp.zeros((q.shape[0], _LW), jnp.float32)
    for j in range(ncol):
        cnt2 = cnt2 + (cols[j] <= t).astype(jnp.float32)
    flags_ref[2] = jnp.max(cnt2).astype(jnp.int32)

    @pl.when(flags_ref[1] > 0)
    def _merge1():
        b_ref[...] = bb
        g_ref[...] = gg
        vv_ref[...] = extract_values(ml)
        run_insertions()

    @pl.when(flags_ref[2] >= 2)
    def _mask1():
        for j in range(ncol):
            dist_ref[:, j * _LW:(j + 1) * _LW] = jnp.where(
                (ml == jnp.float32(j)) & (cols[j] == bb), jnp.inf, cols[j])
        flags_ref[0] = 0

        def round_body(r, carry_r):
            @pl.when(flags_ref[0] == 0)
            def _round():
                cols_r = [dist_ref[:, j * _LW:(j + 1) * _LW]
                          for j in range(ncol)]
                bb_r, ml_r, gg_r = _batch_reduce(cols_r, lane, gbase)
                t_r, ti_r = _top_state(topd_ref, topi_ref)
                cand_r = (bb_r < t_r) | ((bb_r == t_r) & (gg_r < ti_r))
                cnq_r = jnp.sum(cand_r.astype(jnp.float32), axis=1,
                                keepdims=True)
                u_r = jnp.minimum(jnp.max(cnq_r), float(K_NN + 1))
                flags_ref[1] = u_r.astype(jnp.int32)

                @pl.when(u_r == 0.0)
                def _done():
                    flags_ref[0] = 1

                @pl.when(u_r > 0.0)
                def _merge_r():
                    for j in range(ncol):
                        dist_ref[:, j * _LW:(j + 1) * _LW] = jnp.where(
                            (ml_r == jnp.float32(j)) & (cols_r[j] == bb_r),
                            jnp.inf, cols_r[j])
                    b_ref[...] = bb_r
                    g_ref[...] = gg_r
                    vv_ref[...] = extract_values(ml_r)
                    run_insertions()

            return carry_r

        jax.lax.fori_loop(0, flags_ref[2] - 1, round_body, 0)

    @pl.when(c == n_chunks - 1)
    def _emit():
        o_ref[...] = jnp.sum(topv_ref[...], axis=1, keepdims=True) / jnp.float32(K_NN)


def kernel(queries, keys, values):
    n_queries, dim = queries.shape
    n_keys = keys.shape[0]
    chunk = 4096
    n_chunks = pl.cdiv(n_keys, chunk)
    n_pad = n_chunks * chunk

    keys_p = jnp.pad(keys, ((0, n_pad - n_keys), (0, 0)),
                     constant_values=_PAD)
    values_p = jnp.pad(values, (0, n_pad - n_keys)).reshape(n_chunks, 1, chunk)

    out = pl.pallas_call(
        functools.partial(_knn_kernel, n_keys=n_keys, chunk=chunk,
                          n_chunks=n_chunks),
        grid=(n_chunks,),
        in_specs=[
            pl.BlockSpec((n_queries, dim), lambda c: (0, 0)),
            pl.BlockSpec((chunk, dim), lambda c: (c, 0)),
            pl.BlockSpec((1, 1, chunk), lambda c: (c, 0, 0)),
        ],
        out_specs=pl.BlockSpec((n_queries, 1), lambda c: (0, 0)),
        out_shape=jax.ShapeDtypeStruct((n_queries, 1), jnp.float32),
        scratch_shapes=[
            pltpu.VMEM((n_queries, chunk), jnp.float32),
            pltpu.VMEM((n_queries, _LW), jnp.float32),
            pltpu.VMEM((n_queries, _LW), jnp.float32),
            pltpu.VMEM((n_queries, _LW), jnp.float32),
            pltpu.VMEM((n_queries, K_NN), jnp.float32),
            pltpu.VMEM((n_queries, K_NN), jnp.float32),
            pltpu.VMEM((n_queries, K_NN), jnp.float32),
            pltpu.SMEM((4,), jnp.int32),
        ],
    )(queries, keys_p, values_p)
    return out[:, 0]


# speedup vs baseline: 2.6429x; 1.0450x over previous
import functools

import jax
import jax.numpy as jnp
from jax.experimental import pallas as pl
from jax.experimental.pallas import tpu as pltpu

K_NN = 32
_LW = 128
_BIG = float(2 ** 30)
_IBASE = float(2 ** 22)
_PAD = 1.0e15


def _batch_reduce(cols, lane, gbase):
    mins = cols
    while len(mins) > 1:
        nxt = [jnp.minimum(mins[2 * i], mins[2 * i + 1])
               for i in range(len(mins) // 2)]
        if len(mins) % 2:
            nxt.append(mins[-1])
        mins = nxt
    bb = mins[0]
    ml = jnp.full(bb.shape, float(len(cols)), jnp.float32)
    for j in reversed(range(len(cols))):
        ml = jnp.where(cols[j] == bb, jnp.float32(j), ml)
    gg = ml * jnp.float32(_LW) + lane + gbase
    return bb, ml, gg


def _top_state(topd_ref, topi_ref):
    topd = topd_ref[...]
    topi = topi_ref[...]
    t = jnp.max(topd, axis=1, keepdims=True)
    ti = jnp.max(jnp.where(topd == t, topi, -1.0), axis=1, keepdims=True)
    return t, ti


def _knn_kernel(q_ref, k_ref, v_ref, o_ref, dist_ref, b_ref, g_ref, vv_ref,
                topd_ref, topi_ref, topv_ref, flags_ref,
                *, n_keys, chunk, n_chunks):
    c = pl.program_id(0)
    ncol = chunk // _LW

    @pl.when(c == 0)
    def _init():
        topd_ref[...] = jnp.full_like(topd_ref, jnp.inf)
        topi_ref[...] = (jax.lax.broadcasted_iota(jnp.int32, topi_ref.shape, 1)
                         .astype(jnp.float32) + _IBASE)
        topv_ref[...] = jnp.zeros_like(topv_ref)

    q = q_ref[...]
    k = k_ref[...]
    vb = v_ref[0]

    qsq = jnp.sum(q * q, axis=1, keepdims=True)
    ksq = jnp.sum(k * k, axis=1)[None, :]
    qk = jax.lax.dot_general(q, k, (((1,), (1,)), ((), ())),
                             preferred_element_type=jnp.float32)
    dist = qsq - 2.0 * qk + ksq

    lane = jax.lax.broadcasted_iota(jnp.int32, (1, _LW), 1).astype(jnp.float32)
    gbase = jnp.float32(c * chunk)

    def run_insertions():
        ub = (flags_ref[1] + 1) // 2

        def one_insertion():
            bcur = b_ref[...]
            gcur = g_ref[...]
            bm = jnp.min(bcur, axis=1, keepdims=True)
            gm = jnp.min(jnp.where(bcur == bm, gcur, _BIG),
                         axis=1, keepdims=True)
            sel = (bcur == bm) & (gcur == gm)
            val = jnp.sum(jnp.where(sel, vv_ref[...], 0.0),
                          axis=1, keepdims=True)
            td = topd_ref[...]
            tix = topi_ref[...]
            tt = jnp.max(td, axis=1, keepdims=True)
            tti = jnp.max(jnp.where(td == tt, tix, -1.0),
                          axis=1, keepdims=True)
            acc = (bm < tt) | ((bm == tt) & (gm < tti))
            repl = (td == tt) & (tix == tti) & acc
            topd_ref[...] = jnp.where(repl, bm, td)
            topi_ref[...] = jnp.where(repl, gm, tix)
            topv_ref[...] = jnp.where(repl, val, topv_ref[...])
            b_ref[...] = jnp.where(sel, jnp.inf, bcur)

        def ins_body(i, carry_i):
            one_insertion()
            one_insertion()
            return carry_i

        jax.lax.fori_loop(0, ub, ins_body, 0)

    def extract_values(ml):
        vvv = jnp.zeros((q.shape[0], _LW), jnp.float32)
        for j in range(ncol):
            vvv = jnp.where(ml == jnp.float32(j),
                            vb[:, j * _LW:(j + 1) * _LW], vvv)
        return vvv

    cols = [dist[:, j * _LW:(j + 1) * _LW] for j in range(ncol)]
    bb, ml, gg = _batch_reduce(cols, lane, gbase)
    t, ti = _top_state(topd_ref, topi_ref)
    cand = (bb < t) | ((bb == t) & (gg < ti))
    cnq = jnp.sum(cand.astype(jnp.float32), axis=1, keepdims=True)
    flags_ref[1] = jnp.minimum(jnp.max(cnq), float(K_NN + 1)).astype(jnp.int32)
    cnt2 = jnp.zeros((q.shape[0], _LW), jnp.float32)
    for j in range(ncol):
        cnt2 = cnt2 + (cols[j] <= t).astype(jnp.float32)
    flags_ref[2] = jnp.max(cnt2).astype(jnp.int32)

    @pl.when(flags_ref[1] > 0)
    def _merge1():
        b_ref[...] = bb
        g_ref[...] = gg
        vv_ref[...] = extract_values(ml)
        run_insertions()

    @pl.when(flags_ref[2] >= 2)
    def _mask1():
        for j in range(ncol):
            dist_ref[:, j * _LW:(j + 1) * _LW] = jnp.where(
                (ml == jnp.float32(j)) & (cols[j] == bb), jnp.inf, cols[j])
        flags_ref[0] = 0

        def round_body(r, carry_r):
            @pl.when(flags_ref[0] == 0)
            def _round():
                cols_r = [dist_ref[:, j * _LW:(j + 1) * _LW]
                          for j in range(ncol)]
                bb_r, ml_r, gg_r = _batch_reduce(cols_r, lane, gbase)
                t_r, ti_r = _top_state(topd_ref, topi_ref)
                cand_r = (bb_r < t_r) | ((bb_r == t_r) & (gg_r < ti_r))
                cnq_r = jnp.sum(cand_r.astype(jnp.float32), axis=1,
                                keepdims=True)
                u_r = jnp.minimum(jnp.max(cnq_r), float(K_NN + 1))
                flags_ref[1] = u_r.astype(jnp.int32)

                @pl.when(u_r == 0.0)
                def _done():
                    flags_ref[0] = 1

                @pl.when(u_r > 0.0)
                def _merge_r():
                    for j in range(ncol):
                        dist_ref[:, j * _LW:(j + 1) * _LW] = jnp.where(
                            (ml_r == jnp.float32(j)) & (cols_r[j] == bb_r),
                            jnp.inf, cols_r[j])
                    b_ref[...] = bb_r
                    g_ref[...] = gg_r
                    vv_ref[...] = extract_values(ml_r)
                    run_insertions()

            return carry_r

        jax.lax.fori_loop(0, flags_ref[2] - 1, round_body, 0)

    @pl.when(c == n_chunks - 1)
    def _emit():
        o_ref[...] = jnp.sum(topv_ref[...], axis=1, keepdims=True) / jnp.float32(K_NN)


def kernel(queries, keys, values):
    n_queries, dim = queries.shape
    n_keys = keys.shape[0]
    chunk = 4096
    n_chunks = pl.cdiv(n_keys, chunk)
    n_pad = n_chunks * chunk

    keys_p = jnp.pad(keys, ((0, n_pad - n_keys), (0, 0)),
                     constant_values=_PAD)
    values_p = jnp.pad(values, (0, n_pad - n_keys)).reshape(n_chunks, 1, chunk)

    out = pl.pallas_call(
        functools.partial(_knn_kernel, n_keys=n_keys, chunk=chunk,
                          n_chunks=n_chunks),
        grid=(n_chunks,),
        in_specs=[
            pl.BlockSpec((n_queries, dim), lambda c: (0, 0)),
            pl.BlockSpec((chunk, dim), lambda c: (c, 0)),
            pl.BlockSpec((1, 1, chunk), lambda c: (c, 0, 0)),
        ],
        out_specs=pl.BlockSpec((n_queries, 1), lambda c: (0, 0)),
        out_shape=jax.ShapeDtypeStruct((n_queries, 1), jnp.float32),
        scratch_shapes=[
            pltpu.VMEM((n_queries, chunk), jnp.float32),
            pltpu.VMEM((n_queries, _LW), jnp.float32),
            pltpu.VMEM((n_queries, _LW), jnp.float32),
            pltpu.VMEM((n_queries, _LW), jnp.float32),
            pltpu.VMEM((n_queries, K_NN), jnp.float32),
            pltpu.VMEM((n_queries, K_NN), jnp.float32),
            pltpu.VMEM((n_queries, K_NN), jnp.float32),
            pltpu.SMEM((4,), jnp.int32),
        ],
    )(queries, keys_p, values_p)
    return out[:, 0]


# speedup vs baseline: 2.7325x; 1.0339x over previous
import functools

import jax
import jax.numpy as jnp
from jax.experimental import pallas as pl
from jax.experimental.pallas import tpu as pltpu

K_NN = 32
_LW = 128
_BIG = float(2 ** 30)
_IBASE = float(2 ** 22)
_PAD = 1.0e15


def _batch_reduce(cols, lane, gbase):
    mins = cols
    while len(mins) > 1:
        nxt = [jnp.minimum(mins[2 * i], mins[2 * i + 1])
               for i in range(len(mins) // 2)]
        if len(mins) % 2:
            nxt.append(mins[-1])
        mins = nxt
    bb = mins[0]
    ml = jnp.full(bb.shape, float(len(cols)), jnp.float32)
    for j in reversed(range(len(cols))):
        ml = jnp.where(cols[j] == bb, jnp.float32(j), ml)
    gg = ml * jnp.float32(_LW) + lane + gbase
    return bb, ml, gg


def _top_state(topd_ref, topi_ref):
    topd = topd_ref[...]
    topi = topi_ref[...]
    t = jnp.max(topd, axis=1, keepdims=True)
    ti = jnp.max(jnp.where(topd == t, topi, -1.0), axis=1, keepdims=True)
    return t, ti


def _knn_kernel(q_ref, k_ref, v_ref, o_ref, dist_ref, b_ref, g_ref, vv_ref,
                topd_ref, topi_ref, topv_ref, flags_ref,
                *, n_keys, chunk, n_chunks):
    c = pl.program_id(0)
    ncol = chunk // _LW

    @pl.when(c == 0)
    def _init():
        topd_ref[...] = jnp.full_like(topd_ref, jnp.inf)
        topi_ref[...] = (jax.lax.broadcasted_iota(jnp.int32, topi_ref.shape, 1)
                         .astype(jnp.float32) + _IBASE)
        topv_ref[...] = jnp.zeros_like(topv_ref)

    q = q_ref[...]
    k = k_ref[...]
    vb = v_ref[0]

    qsq = jnp.sum(q * q, axis=1, keepdims=True)
    ksq = jnp.sum(k * k, axis=1)[None, :]
    qk = jax.lax.dot_general(q, k, (((1,), (1,)), ((), ())),
                             preferred_element_type=jnp.float32)
    dist = qsq - 2.0 * qk + ksq

    lane = jax.lax.broadcasted_iota(jnp.int32, (1, _LW), 1).astype(jnp.float32)
    gbase = jnp.float32(c * chunk)

    def run_insertions():
        ub = (flags_ref[1] + 1) // 2

        def one_insertion():
            bcur = b_ref[...]
            gcur = g_ref[...]
            bm = jnp.min(bcur, axis=1, keepdims=True)
            gm = jnp.min(jnp.where(bcur == bm, gcur, _BIG),
                         axis=1, keepdims=True)
            sel = (bcur == bm) & (gcur == gm)
            val = jnp.sum(jnp.where(sel, vv_ref[...], 0.0),
                          axis=1, keepdims=True)
            td = topd_ref[...]
            tix = topi_ref[...]
            tt = jnp.max(td, axis=1, keepdims=True)
            tti = jnp.max(jnp.where(td == tt, tix, -1.0),
                          axis=1, keepdims=True)
            acc = (bm < tt) | ((bm == tt) & (gm < tti))
            repl = (td == tt) & (tix == tti) & acc
            topd_ref[...] = jnp.where(repl, bm, td)
            topi_ref[...] = jnp.where(repl, gm, tix)
            topv_ref[...] = jnp.where(repl, val, topv_ref[...])
            b_ref[...] = jnp.where(sel, jnp.inf, bcur)

        def ins_body(i, carry_i):
            one_insertion()
            one_insertion()
            return carry_i

        jax.lax.fori_loop(0, ub, ins_body, 0)

    def extract_values(ml):
        vvv = jnp.zeros((q.shape[0], _LW), jnp.float32)
        for j in range(ncol):
            vvv = jnp.where(ml == jnp.float32(j),
                            vb[:, j * _LW:(j + 1) * _LW], vvv)
        return vvv

    cols = [dist[:, j * _LW:(j + 1) * _LW] for j in range(ncol)]
    bb, ml, gg = _batch_reduce(cols, lane, gbase)
    t, ti = _top_state(topd_ref, topi_ref)
    cand = (bb < t) | ((bb == t) & (gg < ti))
    cnq = jnp.sum(cand.astype(jnp.float32), axis=1, keepdims=True)
    flags_ref[1] = jnp.minimum(jnp.max(cnq), float(K_NN + 1)).astype(jnp.int32)
    cnt2 = jnp.zeros((q.shape[0], _LW), jnp.float32)
    for j in range(ncol):
        cnt2 = cnt2 + (cols[j] <= t).astype(jnp.float32)
    flags_ref[2] = jnp.max(cnt2).astype(jnp.int32)

    @pl.when(flags_ref[1] > 0)
    def _merge1():
        b_ref[...] = bb
        g_ref[...] = gg
        vv_ref[...] = extract_values(ml)
        run_insertions()

    @pl.when(flags_ref[2] >= 2)
    def _mask1():
        for j in range(ncol):
            dist_ref[:, j * _LW:(j + 1) * _LW] = jnp.where(
                (ml == jnp.float32(j)) & (cols[j] == bb), jnp.inf, cols[j])
        flags_ref[0] = 0

        def round_body(r, carry_r):
            @pl.when(flags_ref[0] == 0)
            def _round():
                cols_r = [dist_ref[:, j * _LW:(j + 1) * _LW]
                          for j in range(ncol)]
                bb_r, ml_r, gg_r = _batch_reduce(cols_r, lane, gbase)
                t_r, ti_r = _top_state(topd_ref, topi_ref)
                cand_r = (bb_r < t_r) | ((bb_r == t_r) & (gg_r < ti_r))
                cnq_r = jnp.sum(cand_r.astype(jnp.float32), axis=1,
                                keepdims=True)
                u_r = jnp.minimum(jnp.max(cnq_r), float(K_NN + 1))
                flags_ref[1] = u_r.astype(jnp.int32)

                @pl.when(u_r == 0.0)
                def _done():
                    flags_ref[0] = 1

                @pl.when(u_r > 0.0)
                def _merge_r():
                    for j in range(ncol):
                        dist_ref[:, j * _LW:(j + 1) * _LW] = jnp.where(
                            (ml_r == jnp.float32(j)) & (cols_r[j] == bb_r),
                            jnp.inf, cols_r[j])
                    b_ref[...] = bb_r
                    g_ref[...] = gg_r
                    vv_ref[...] = extract_values(ml_r)
                    run_insertions()

            return carry_r

        jax.lax.fori_loop(0, flags_ref[2] - 1, round_body, 0)

    @pl.when(c == n_chunks - 1)
    def _emit():
        o_ref[...] = jnp.sum(topv_ref[...], axis=1, keepdims=True) / jnp.float32(K_NN)


def kernel(queries, keys, values):
    n_queries, dim = queries.shape
    n_keys = keys.shape[0]
    chunk = 6144
    n_chunks = pl.cdiv(n_keys, chunk)
    n_pad = n_chunks * chunk

    keys_p = jnp.pad(keys, ((0, n_pad - n_keys), (0, 0)),
                     constant_values=_PAD)
    values_p = jnp.pad(values, (0, n_pad - n_keys)).reshape(n_chunks, 1, chunk)

    out = pl.pallas_call(
        functools.partial(_knn_kernel, n_keys=n_keys, chunk=chunk,
                          n_chunks=n_chunks),
        grid=(n_chunks,),
        in_specs=[
            pl.BlockSpec((n_queries, dim), lambda c: (0, 0)),
            pl.BlockSpec((chunk, dim), lambda c: (c, 0)),
            pl.BlockSpec((1, 1, chunk), lambda c: (c, 0, 0)),
        ],
        out_specs=pl.BlockSpec((n_queries, 1), lambda c: (0, 0)),
        out_shape=jax.ShapeDtypeStruct((n_queries, 1), jnp.float32),
        scratch_shapes=[
            pltpu.VMEM((n_queries, chunk), jnp.float32),
            pltpu.VMEM((n_queries, _LW), jnp.float32),
            pltpu.VMEM((n_queries, _LW), jnp.float32),
            pltpu.VMEM((n_queries, _LW), jnp.float32),
            pltpu.VMEM((n_queries, K_NN), jnp.float32),
            pltpu.VMEM((n_queries, K_NN), jnp.float32),
            pltpu.VMEM((n_queries, K_NN), jnp.float32),
            pltpu.SMEM((4,), jnp.int32),
        ],
    )(queries, keys_p, values_p)
    return out[:, 0]
